# initial kernel scaffold (unmeasured)
import jax
import jax.numpy as jnp
from jax import lax
from jax.experimental import pallas as pl
from jax.experimental.pallas import tpu as pltpu

B, H, D, BS = 8, 8, 128, 16
NP = 512
NK = NP * BS
HD = H * D
PB = 64
KBLK = PB * BS
NB = NP // PB
SCALE = D ** -0.5
NEG_INF = -1e30


def _body(q_ref, k_ref, v_ref, bt_ref, lens_ref, out_ref,
          macc_ref, lacc_ref, oacc_ref, orecv_ref,
          ssend_ref, srecv_ref, send_sems, recv_sems):
    g = pl.program_id(0)
    my_x = lax.axis_index("x")
    my_y = lax.axis_index("y")

    @pl.when(g == 0)
    def _init():
        macc_ref[...] = jnp.full((B, H), NEG_INF, jnp.float32)
        lacc_ref[...] = jnp.zeros((B, H), jnp.float32)
        oacc_ref[...] = jnp.zeros((B, HD), jnp.float32)

    bt = bt_ref[...]
    lens = lens_ref[...]
    valid = lax.broadcasted_iota(jnp.int32, (B, NP), 1) < lens
    pidc = (my_y * NP + g * PB
            + lax.broadcasted_iota(jnp.int32, (B, PB, NP), 1))
    eqc = (bt[:, None, :] == pidc) & valid[:, None, :]
    wc = jnp.sum(eqc.astype(jnp.float32), axis=2)
    expand = (lax.broadcasted_iota(jnp.int32, (PB, KBLK), 1) // BS
              == lax.broadcasted_iota(jnp.int32, (PB, KBLK), 0))
    wkc = lax.dot_general(
        wc.astype(jnp.bfloat16), expand.astype(jnp.bfloat16),
        (((1,), (0,)), ((), ())),
        preferred_element_type=jnp.float32)

    qb = (q_ref[...] * SCALE).astype(jnp.bfloat16)
    kb = k_ref[...].astype(jnp.bfloat16)
    vb = v_ref[...].astype(jnp.bfloat16)

    for h in range(H):
        sl = slice(h * D, (h + 1) * D)
        s = lax.dot_general(qb[:, sl], kb[:, sl],
                            (((1,), (1,)), ((), ())),
                            preferred_element_type=jnp.float32)
        m_prev = macc_ref[:, h:h + 1]
        m_new = jnp.maximum(m_prev, jnp.max(s, axis=1, keepdims=True))
        alpha = jnp.exp(m_prev - m_new)
        p = jnp.exp(s - m_new) * wkc
        lacc_ref[:, h:h + 1] = (lacc_ref[:, h:h + 1] * alpha
                                + jnp.sum(p, axis=1, keepdims=True))
        pv = lax.dot_general(p.astype(jnp.bfloat16), vb[:, sl],
                             (((1,), (0,)), ((), ())),
                             preferred_element_type=jnp.float32)
        oacc_ref[:, sl] = oacc_ref[:, sl] * alpha + pv
        macc_ref[:, h:h + 1] = m_new

    @pl.when(g == NB - 1)
    def _exchange():
        ssend_ref[0, :, :] = macc_ref[...]
        ssend_ref[1, :, :] = lacc_ref[...]
        peer = (my_x, 1 - my_y)
        rdma_o = pltpu.make_async_remote_copy(
            src_ref=oacc_ref, dst_ref=orecv_ref,
            send_sem=send_sems.at[0], recv_sem=recv_sems.at[0],
            device_id=peer, device_id_type=pl.DeviceIdType.MESH)
        rdma_s = pltpu.make_async_remote_copy(
            src_ref=ssend_ref, dst_ref=srecv_ref,
            send_sem=send_sems.at[1], recv_sem=recv_sems.at[1],
            device_id=peer, device_id_type=pl.DeviceIdType.MESH)
        rdma_o.start()
        rdma_s.start()
        rdma_o.wait()
        rdma_s.wait()

        m_a = macc_ref[...]
        l_a = lacc_ref[...]
        m_b = srecv_ref[0, :, :]
        l_b = srecv_ref[1, :, :]
        mm = jnp.maximum(m_a, m_b)
        ea = jnp.exp(m_a - mm)
        eb = jnp.exp(m_b - mm)
        denom = l_a * ea + l_b * eb
        ca = ea / denom
        cb = eb / denom
        for h in range(H):
            sl = slice(h * D, (h + 1) * D)
            out_ref[:, sl] = (oacc_ref[:, sl] * ca[:, h:h + 1]
                              + orecv_ref[:, sl] * cb[:, h:h + 1])


def kernel(Q, K, V, bt, lens):
    q2 = Q.reshape(B, HD)
    k2 = K.reshape(NK, HD)
    v2 = V.reshape(NK, HD)
    lens2 = lens.reshape(B, 1)
    out2 = pl.pallas_call(
        _body,
        grid=(NB,),
        out_shape=jax.ShapeDtypeStruct((B, HD), jnp.float32),
        in_specs=[
            pl.BlockSpec((B, HD), lambda g: (0, 0)),
            pl.BlockSpec((KBLK, HD), lambda g: (g, 0)),
            pl.BlockSpec((KBLK, HD), lambda g: (g, 0)),
            pl.BlockSpec((B, NP), lambda g: (0, 0)),
            pl.BlockSpec((B, 1), lambda g: (0, 0)),
        ],
        out_specs=pl.BlockSpec((B, HD), lambda g: (0, 0)),
        scratch_shapes=[
            pltpu.VMEM((B, H), jnp.float32),
            pltpu.VMEM((B, H), jnp.float32),
            pltpu.VMEM((B, HD), jnp.float32),
            pltpu.VMEM((B, HD), jnp.float32),
            pltpu.VMEM((2, B, H), jnp.float32),
            pltpu.VMEM((2, B, H), jnp.float32),
            pltpu.SemaphoreType.DMA((2,)),
            pltpu.SemaphoreType.DMA((2,)),
        ],
        compiler_params=pltpu.CompilerParams(
            collective_id=0,
            dimension_semantics=("arbitrary",),
        ),
    )(q2, k2, v2, bt, lens2)
    return out2.reshape(B, 1, H, D)


# baseline (device time: 87463 ns/iter reference)
import jax
import jax.numpy as jnp
from jax import lax
from jax.experimental import pallas as pl
from jax.experimental.pallas import tpu as pltpu

B, H, D, BS = 8, 8, 128, 16
NP = 512
NK = NP * BS
HD = H * D
PB = 64
KBLK = PB * BS
NB = NP // PB
SCALE = D ** -0.5
NEG_INF = -1e30


def _body(q_ref, k_ref, v_ref, bt_ref, lens_ref, out_ref,
          macc_ref, lacc_ref, oacc_ref, orecv_ref,
          ssend_ref, srecv_ref, send_sems, recv_sems):
    g = pl.program_id(0)
    my_x = lax.axis_index("x")
    my_y = lax.axis_index("y")

    @pl.when(g == 0)
    def _init():
        macc_ref[...] = jnp.full((B, H), NEG_INF, jnp.float32)
        lacc_ref[...] = jnp.zeros((B, H), jnp.float32)
        oacc_ref[...] = jnp.zeros((B, HD), jnp.float32)

    bt = bt_ref[...]
    lens = lens_ref[...]
    valid = lax.broadcasted_iota(jnp.int32, (B, NP), 1) < lens
    pidc = (my_y * NP + g * PB
            + lax.broadcasted_iota(jnp.int32, (B, PB, NP), 1))
    eqc = (bt[:, None, :] == pidc) & valid[:, None, :]
    wc = jnp.sum(eqc.astype(jnp.float32), axis=2)
    expand = (lax.broadcasted_iota(jnp.int32, (PB, KBLK), 1) // BS
              == lax.broadcasted_iota(jnp.int32, (PB, KBLK), 0))
    wkc = lax.dot_general(
        wc.astype(jnp.bfloat16), expand.astype(jnp.bfloat16),
        (((1,), (0,)), ((), ())),
        preferred_element_type=jnp.float32)

    qb = (q_ref[...] * SCALE).astype(jnp.bfloat16)
    kb = k_ref[...].astype(jnp.bfloat16)
    vb = v_ref[...].astype(jnp.bfloat16)

    for h in range(H):
        sl = slice(h * D, (h + 1) * D)
        s = lax.dot_general(qb[:, sl], kb[:, sl],
                            (((1,), (1,)), ((), ())),
                            preferred_element_type=jnp.float32)
        m_prev = macc_ref[:, h:h + 1]
        m_new = jnp.maximum(m_prev, jnp.max(s, axis=1, keepdims=True))
        alpha = jnp.exp(m_prev - m_new)
        p = jnp.exp(s - m_new) * wkc
        lacc_ref[:, h:h + 1] = (lacc_ref[:, h:h + 1] * alpha
                                + jnp.sum(p, axis=1, keepdims=True))
        pv = lax.dot_general(p.astype(jnp.bfloat16), vb[:, sl],
                             (((1,), (0,)), ((), ())),
                             preferred_element_type=jnp.float32)
        oacc_ref[:, sl] = oacc_ref[:, sl] * alpha + pv
        macc_ref[:, h:h + 1] = m_new

    @pl.when(g == NB - 1)
    def _exchange():
        ssend_ref[0, :, :] = macc_ref[...]
        ssend_ref[1, :, :] = lacc_ref[...]
        peer = (my_x, 1 - my_y)
        rdma_o = pltpu.make_async_remote_copy(
            src_ref=oacc_ref, dst_ref=orecv_ref,
            send_sem=send_sems.at[0], recv_sem=recv_sems.at[0],
            device_id=peer, device_id_type=pl.DeviceIdType.MESH)
        rdma_s = pltpu.make_async_remote_copy(
            src_ref=ssend_ref, dst_ref=srecv_ref,
            send_sem=send_sems.at[1], recv_sem=recv_sems.at[1],
            device_id=peer, device_id_type=pl.DeviceIdType.MESH)
        rdma_o.start()
        rdma_s.start()
        rdma_o.wait()
        rdma_s.wait()

        m_a = macc_ref[...]
        l_a = lacc_ref[...]
        m_b = srecv_ref[0, :, :]
        l_b = srecv_ref[1, :, :]
        mm = jnp.maximum(m_a, m_b)
        ea = jnp.exp(m_a - mm)
        eb = jnp.exp(m_b - mm)
        denom = l_a * ea + l_b * eb
        ca = ea / denom
        cb = eb / denom
        for h in range(H):
            sl = slice(h * D, (h + 1) * D)
            out_ref[:, sl] = (oacc_ref[:, sl] * ca[:, h:h + 1]
                              + orecv_ref[:, sl] * cb[:, h:h + 1])


def kernel(Q, K, V, bt, lens):
    q2 = Q.reshape(B, HD)
    k2 = K.reshape(NK, HD)
    v2 = V.reshape(NK, HD)
    lens2 = lens.reshape(B, 1)
    out2 = pl.pallas_call(
        _body,
        grid=(NB,),
        out_shape=jax.ShapeDtypeStruct((B, HD), jnp.float32),
        in_specs=[
            pl.BlockSpec((B, HD), lambda g: (0, 0)),
            pl.BlockSpec((KBLK, HD), lambda g: (g, 0)),
            pl.BlockSpec((KBLK, HD), lambda g: (g, 0)),
            pl.BlockSpec((B, NP), lambda g: (0, 0)),
            pl.BlockSpec((B, 1), lambda g: (0, 0)),
        ],
        out_specs=pl.BlockSpec((B, HD), lambda g: (0, 0)),
        scratch_shapes=[
            pltpu.VMEM((B, H), jnp.float32),
            pltpu.VMEM((B, H), jnp.float32),
            pltpu.VMEM((B, HD), jnp.float32),
            pltpu.VMEM((B, HD), jnp.float32),
            pltpu.VMEM((2, B, H), jnp.float32),
            pltpu.VMEM((2, B, H), jnp.float32),
            pltpu.SemaphoreType.DMA((2,)),
            pltpu.SemaphoreType.DMA((2,)),
        ],
        compiler_params=pltpu.CompilerParams(
            dimension_semantics=("arbitrary",),
        ),
    )(q2, k2, v2, bt, lens2)
    return out2.reshape(B, 1, H, D)


# device time: 44566 ns/iter; 1.9625x vs baseline; 1.9625x over previous
import jax
import jax.numpy as jnp
from jax import lax
from jax.experimental import pallas as pl
from jax.experimental.pallas import tpu as pltpu

B, H, D, BS = 8, 8, 128, 16
NP = 512
NK = NP * BS
HD = H * D
PB = 64
KBLK = PB * BS
NB = NP // PB
SCALE = D ** -0.5
NEG_INF = -1e30


def _body(q_ref, k_ref, v_ref, bt_ref, lens_ref, out_ref,
          macc_ref, lacc_ref, oacc_ref, orecv_ref,
          ssend_ref, srecv_ref, send_sems, recv_sems):
    g = pl.program_id(0)
    my_x = lax.axis_index("x")
    my_y = lax.axis_index("y")

    @pl.when(g == 0)
    def _init():
        macc_ref[...] = jnp.full((B, H), NEG_INF, jnp.float32)
        lacc_ref[...] = jnp.zeros((B, H), jnp.float32)
        oacc_ref[...] = jnp.zeros((B, HD), jnp.float32)

    bt = bt_ref[...]
    lens = lens_ref[...]
    valid = lax.broadcasted_iota(jnp.int32, (B, NP), 1) < lens
    pidc = (my_y * NP + g * PB
            + lax.broadcasted_iota(jnp.int32, (B, PB, NP), 1))
    eqc = (bt[:, None, :] == pidc) & valid[:, None, :]
    wc = jnp.sum(eqc.astype(jnp.float32), axis=2)
    expand = (lax.broadcasted_iota(jnp.int32, (PB, KBLK), 1) // BS
              == lax.broadcasted_iota(jnp.int32, (PB, KBLK), 0))
    wkc = lax.dot_general(
        wc.astype(jnp.bfloat16), expand.astype(jnp.bfloat16),
        (((1,), (0,)), ((), ())),
        preferred_element_type=jnp.float32)

    qb = (q_ref[...] * SCALE).astype(jnp.bfloat16)
    kb = k_ref[...].astype(jnp.bfloat16).reshape(KBLK, HD)
    vb = v_ref[...].astype(jnp.bfloat16).reshape(KBLK, HD)

    for h in range(H):
        sl = slice(h * D, (h + 1) * D)
        s = lax.dot_general(qb[:, sl], kb[:, sl],
                            (((1,), (1,)), ((), ())),
                            preferred_element_type=jnp.float32)
        m_prev = macc_ref[:, h:h + 1]
        m_new = jnp.maximum(m_prev, jnp.max(s, axis=1, keepdims=True))
        alpha = jnp.exp(m_prev - m_new)
        p = jnp.exp(s - m_new) * wkc
        lacc_ref[:, h:h + 1] = (lacc_ref[:, h:h + 1] * alpha
                                + jnp.sum(p, axis=1, keepdims=True))
        pv = lax.dot_general(p.astype(jnp.bfloat16), vb[:, sl],
                             (((1,), (0,)), ((), ())),
                             preferred_element_type=jnp.float32)
        oacc_ref[:, sl] = oacc_ref[:, sl] * alpha + pv
        macc_ref[:, h:h + 1] = m_new

    @pl.when(g == NB - 1)
    def _exchange():
        ssend_ref[0, :, :] = macc_ref[...]
        ssend_ref[1, :, :] = lacc_ref[...]
        peer = (my_x, 1 - my_y)
        rdma_o = pltpu.make_async_remote_copy(
            src_ref=oacc_ref, dst_ref=orecv_ref,
            send_sem=send_sems.at[0], recv_sem=recv_sems.at[0],
            device_id=peer, device_id_type=pl.DeviceIdType.MESH)
        rdma_s = pltpu.make_async_remote_copy(
            src_ref=ssend_ref, dst_ref=srecv_ref,
            send_sem=send_sems.at[1], recv_sem=recv_sems.at[1],
            device_id=peer, device_id_type=pl.DeviceIdType.MESH)
        rdma_o.start()
        rdma_s.start()
        rdma_o.wait()
        rdma_s.wait()

        m_a = macc_ref[...]
        l_a = lacc_ref[...]
        m_b = srecv_ref[0, :, :]
        l_b = srecv_ref[1, :, :]
        mm = jnp.maximum(m_a, m_b)
        ea = jnp.exp(m_a - mm)
        eb = jnp.exp(m_b - mm)
        denom = l_a * ea + l_b * eb
        ca = ea / denom
        cb = eb / denom
        for h in range(H):
            sl = slice(h * D, (h + 1) * D)
            out_ref[:, sl] = (oacc_ref[:, sl] * ca[:, h:h + 1]
                              + orecv_ref[:, sl] * cb[:, h:h + 1])


def kernel(Q, K, V, bt, lens):
    q2 = Q.reshape(B, HD)
    k2 = K.reshape(NK, H, D)
    v2 = V.reshape(NK, H, D)
    lens2 = lens.reshape(B, 1)
    out2 = pl.pallas_call(
        _body,
        grid=(NB,),
        out_shape=jax.ShapeDtypeStruct((B, HD), jnp.float32),
        in_specs=[
            pl.BlockSpec((B, HD), lambda g: (0, 0)),
            pl.BlockSpec((KBLK, H, D), lambda g: (g, 0, 0)),
            pl.BlockSpec((KBLK, H, D), lambda g: (g, 0, 0)),
            pl.BlockSpec((B, NP), lambda g: (0, 0)),
            pl.BlockSpec((B, 1), lambda g: (0, 0)),
        ],
        out_specs=pl.BlockSpec((B, HD), lambda g: (0, 0)),
        scratch_shapes=[
            pltpu.VMEM((B, H), jnp.float32),
            pltpu.VMEM((B, H), jnp.float32),
            pltpu.VMEM((B, HD), jnp.float32),
            pltpu.VMEM((B, HD), jnp.float32),
            pltpu.VMEM((2, B, H), jnp.float32),
            pltpu.VMEM((2, B, H), jnp.float32),
            pltpu.SemaphoreType.DMA((2,)),
            pltpu.SemaphoreType.DMA((2,)),
        ],
        compiler_params=pltpu.CompilerParams(
            dimension_semantics=("arbitrary",),
        ),
    )(q2, k2, v2, bt, lens2)
    return out2.reshape(B, 1, H, D)


# device time: 36184 ns/iter; 2.4172x vs baseline; 1.2316x over previous
import jax
import jax.numpy as jnp
from jax import lax
from jax.experimental import pallas as pl
from jax.experimental.pallas import tpu as pltpu

B, H, D, BS = 8, 8, 128, 16
NP = 512
NK = NP * BS
HD = H * D
PB = 64
KBLK = PB * BS
NB = NP // PB
SCALE = D ** -0.5
NEG_INF = -1e30


BH = B * H


def _body(q_ref, k_ref, v_ref, bt_ref, lens_ref, out_ref,
          qbd_ref, macc_ref, lacc_ref, oacc_ref, orecv_ref,
          ssend_ref, srecv_ref, send_sems, recv_sems):
    g = pl.program_id(0)
    my_x = lax.axis_index("x")
    my_y = lax.axis_index("y")

    @pl.when(g == 0)
    def _init():
        macc_ref[...] = jnp.full((BH, 1), NEG_INF, jnp.float32)
        lacc_ref[...] = jnp.zeros((BH, 1), jnp.float32)
        oacc_ref[...] = jnp.zeros((B, HD), jnp.float32)
        q64 = jnp.broadcast_to((q_ref[...] * SCALE)[None, :, :],
                               (H, B, HD)).reshape(BH, HD)
        rowh = lax.broadcasted_iota(jnp.int32, (BH, HD), 0) // B
        colh = lax.broadcasted_iota(jnp.int32, (BH, HD), 1) // D
        qbd_ref[...] = jnp.where(rowh == colh, q64, 0.0).astype(jnp.bfloat16)

    bt = bt_ref[...]
    lens = lens_ref[...]
    valid = lax.broadcasted_iota(jnp.int32, (B, NP), 1) < lens
    pidc = (my_y * NP + g * PB
            + lax.broadcasted_iota(jnp.int32, (B, PB, NP), 1))
    eqc = (bt[:, None, :] == pidc) & valid[:, None, :]
    wc = jnp.sum(eqc.astype(jnp.float32), axis=2)
    expand = (lax.broadcasted_iota(jnp.int32, (PB, KBLK), 1) // BS
              == lax.broadcasted_iota(jnp.int32, (PB, KBLK), 0))
    wkc = lax.dot_general(
        wc.astype(jnp.bfloat16), expand.astype(jnp.bfloat16),
        (((1,), (0,)), ((), ())),
        preferred_element_type=jnp.float32)

    kb = k_ref[...].astype(jnp.bfloat16).reshape(KBLK, HD)
    vb = v_ref[...].astype(jnp.bfloat16).reshape(KBLK, HD)

    s_all = lax.dot_general(qbd_ref[...], kb,
                            (((1,), (1,)), ((), ())),
                            preferred_element_type=jnp.float32)
    wk64 = jnp.broadcast_to(wkc[None, :, :], (H, B, KBLK)).reshape(BH, KBLK)
    m_prev = macc_ref[...]
    m_new = jnp.maximum(m_prev, jnp.max(s_all, axis=1, keepdims=True))
    alpha = jnp.exp(m_prev - m_new)
    p = jnp.exp(s_all - m_new) * wk64
    lacc_ref[...] = lacc_ref[...] * alpha + jnp.sum(p, axis=1, keepdims=True)
    macc_ref[...] = m_new
    pb = p.astype(jnp.bfloat16)
    for h in range(H):
        sl = slice(h * D, (h + 1) * D)
        rs = slice(h * B, (h + 1) * B)
        pv = lax.dot_general(pb[rs, :], vb[:, sl],
                             (((1,), (0,)), ((), ())),
                             preferred_element_type=jnp.float32)
        oacc_ref[:, sl] = oacc_ref[:, sl] * alpha[rs, :] + pv

    @pl.when(g == NB - 1)
    def _exchange():
        ssend_ref[0, :, :] = macc_ref[...]
        ssend_ref[1, :, :] = lacc_ref[...]
        peer = (my_x, 1 - my_y)
        rdma_o = pltpu.make_async_remote_copy(
            src_ref=oacc_ref, dst_ref=orecv_ref,
            send_sem=send_sems.at[0], recv_sem=recv_sems.at[0],
            device_id=peer, device_id_type=pl.DeviceIdType.MESH)
        rdma_s = pltpu.make_async_remote_copy(
            src_ref=ssend_ref, dst_ref=srecv_ref,
            send_sem=send_sems.at[1], recv_sem=recv_sems.at[1],
            device_id=peer, device_id_type=pl.DeviceIdType.MESH)
        rdma_o.start()
        rdma_s.start()
        rdma_o.wait()
        rdma_s.wait()

        m_a = macc_ref[...]
        l_a = lacc_ref[...]
        m_b = srecv_ref[0, :, :]
        l_b = srecv_ref[1, :, :]
        mm = jnp.maximum(m_a, m_b)
        ea = jnp.exp(m_a - mm)
        eb = jnp.exp(m_b - mm)
        denom = l_a * ea + l_b * eb
        ca = ea / denom
        cb = eb / denom
        for h in range(H):
            sl = slice(h * D, (h + 1) * D)
            rs = slice(h * B, (h + 1) * B)
            out_ref[:, sl] = (oacc_ref[:, sl] * ca[rs, :]
                              + orecv_ref[:, sl] * cb[rs, :])


def kernel(Q, K, V, bt, lens):
    q2 = Q.reshape(B, HD)
    k2 = K.reshape(NK, H, D)
    v2 = V.reshape(NK, H, D)
    lens2 = lens.reshape(B, 1)
    out2 = pl.pallas_call(
        _body,
        grid=(NB,),
        out_shape=jax.ShapeDtypeStruct((B, HD), jnp.float32),
        in_specs=[
            pl.BlockSpec((B, HD), lambda g: (0, 0)),
            pl.BlockSpec((KBLK, H, D), lambda g: (g, 0, 0)),
            pl.BlockSpec((KBLK, H, D), lambda g: (g, 0, 0)),
            pl.BlockSpec((B, NP), lambda g: (0, 0)),
            pl.BlockSpec((B, 1), lambda g: (0, 0)),
        ],
        out_specs=pl.BlockSpec((B, HD), lambda g: (0, 0)),
        scratch_shapes=[
            pltpu.VMEM((BH, HD), jnp.bfloat16),
            pltpu.VMEM((BH, 1), jnp.float32),
            pltpu.VMEM((BH, 1), jnp.float32),
            pltpu.VMEM((B, HD), jnp.float32),
            pltpu.VMEM((B, HD), jnp.float32),
            pltpu.VMEM((2, BH, 1), jnp.float32),
            pltpu.VMEM((2, BH, 1), jnp.float32),
            pltpu.SemaphoreType.DMA((2,)),
            pltpu.SemaphoreType.DMA((2,)),
        ],
        compiler_params=pltpu.CompilerParams(
            dimension_semantics=("arbitrary",),
        ),
    )(q2, k2, v2, bt, lens2)
    return out2.reshape(B, 1, H, D)
